# baseline (speedup 1.0000x reference)
import jax
import jax.numpy as jnp
from jax.experimental import pallas as pl
from jax.experimental.pallas import tpu as pltpu


def _selector_body(x_ref, w_ref, b_ref, idx_ref):
    pooled = jnp.mean(x_ref[...], axis=0, keepdims=True)
    logits = jnp.dot(pooled, w_ref[...],
                     preferred_element_type=jnp.float32) + b_ref[...]
    l = logits[0]
    iota = jax.lax.iota(jnp.int32, l.shape[0])
    i0 = jnp.argmax(l).astype(jnp.int32)
    l2 = jnp.where(iota == i0, -jnp.inf, l)
    i1 = jnp.argmax(l2).astype(jnp.int32)
    idx_ref[0] = i0
    idx_ref[1] = i1


def _chain_body(idx_ref, x_ref, g_ref, beta_ref, w1_ref, b1_ref, w2_ref,
                b2_ref, t_ref, o_ref, h_ref, w1g_ref, w2s_ref, b1p_ref,
                c2_ref):
    i = pl.program_id(0)
    t = pl.program_id(1)
    R = x_ref.shape[0]
    D = x_ref.shape[1]

    @pl.when(t == 0)
    def _():
        g_col = g_ref[0].reshape(D, 1)
        C = D // 4
        for c in range(4):
            lo, hi = c * C, (c + 1) * C
            w1g_ref[lo:hi, :] = (g_col[lo:hi, :]
                                 * w1_ref[0, lo:hi, :]).astype(jnp.bfloat16)
            w2s_ref[lo:hi, :] = (0.3 * w2_ref[0, lo:hi, :]).astype(jnp.bfloat16)
        b1p_ref[...] = (jnp.dot(beta_ref[0], w1_ref[0],
                                preferred_element_type=jnp.float32)
                        + b1_ref[0])
        c2_ref[...] = 0.3 * b2_ref[0]

    h = jax.lax.cond(
        i == 0,
        lambda: x_ref[...],
        lambda: h_ref[pl.ds(t * R, R), :].astype(jnp.float32))
    mu = jnp.mean(h, axis=1, keepdims=True)
    var = jnp.mean((h - mu) ** 2, axis=1, keepdims=True)
    z = ((h - mu) * jax.lax.rsqrt(var + 1e-5)).astype(jnp.bfloat16)
    a = jnp.tanh(jnp.dot(z, w1g_ref[...],
                         preferred_element_type=jnp.float32) + b1p_ref[...])
    q = jnp.dot(a.astype(jnp.bfloat16), w2s_ref[...],
                preferred_element_type=jnp.float32) + c2_ref[...]
    nh = h + q * (1.0 + 0.1 * t_ref[0])

    @pl.when(i == 0)
    def _():
        h_ref[pl.ds(t * R, R), :] = nh.astype(jnp.bfloat16)

    @pl.when(i == 1)
    def _():
        o_ref[...] = nh


def kernel(embodied_input, disembodied_input, torsion_field, sel_W, sel_b,
           ln_g, ln_beta, W1, b1, W2, b2, max_active_blocks):
    B, S, D = embodied_input.shape
    NB = sel_b.shape[0]
    BS = B * S

    x0 = embodied_input[0]
    top_idx = pl.pallas_call(
        _selector_body,
        in_specs=[
            pl.BlockSpec(memory_space=pltpu.VMEM),
            pl.BlockSpec(memory_space=pltpu.VMEM),
            pl.BlockSpec(memory_space=pltpu.VMEM),
        ],
        out_specs=pl.BlockSpec(memory_space=pltpu.SMEM),
        out_shape=jax.ShapeDtypeStruct((2,), jnp.int32),
    )(x0, sel_W, sel_b.reshape(1, NB))

    R = 1024
    T = BS // R
    S_per_batch = S

    def widx3(i, t, s):
        del t
        return (s[i], 0, 0)

    h = pl.pallas_call(
        _chain_body,
        grid_spec=pltpu.PrefetchScalarGridSpec(
            num_scalar_prefetch=1,
            grid=(2, T),
            in_specs=[
                pl.BlockSpec((R, D),
                             lambda i, t, s: (jnp.where(i == 0, t, T - 1), 0)),
                pl.BlockSpec((1, 1, D), widx3),
                pl.BlockSpec((1, 1, D), widx3),
                pl.BlockSpec((1, D, D), widx3,
                             pipeline_mode=pl.Buffered(buffer_count=1)),
                pl.BlockSpec((1, 1, D), widx3),
                pl.BlockSpec((1, D, D), widx3,
                             pipeline_mode=pl.Buffered(buffer_count=1)),
                pl.BlockSpec((1, 1, D), widx3),
                pl.BlockSpec((1, 1, D),
                             lambda i, t, s: (t * R // S_per_batch, 0, 0)),
            ],
            out_specs=pl.BlockSpec((R, D),
                                   lambda i, t, s: (jnp.where(i == 1, t, 0), 0)),
            scratch_shapes=[
                pltpu.VMEM((BS, D), jnp.bfloat16),
                pltpu.VMEM((D, D), jnp.bfloat16),
                pltpu.VMEM((D, D), jnp.bfloat16),
                pltpu.VMEM((1, D), jnp.float32),
                pltpu.VMEM((1, D), jnp.float32),
            ],
        ),
        out_shape=jax.ShapeDtypeStruct((BS, D), jnp.float32),
        compiler_params=pltpu.CompilerParams(
            dimension_semantics=("arbitrary", "arbitrary"),
        ),
    )(top_idx, embodied_input.reshape(BS, D), ln_g.reshape(NB, 1, D),
      ln_beta.reshape(NB, 1, D), W1, b1.reshape(NB, 1, D), W2,
      b2.reshape(NB, 1, D), torsion_field.reshape(B, 1, D))
    return h.reshape(B, S, D)


# speedup vs baseline: 1.2125x; 1.2125x over previous
import jax
import jax.numpy as jnp
from jax.experimental import pallas as pl
from jax.experimental.pallas import tpu as pltpu


def _selector_body(x_ref, w_ref, b_ref, idx_ref):
    pooled = jnp.mean(x_ref[...], axis=0, keepdims=True)
    logits = jnp.dot(pooled, w_ref[...],
                     preferred_element_type=jnp.float32) + b_ref[...]
    l = logits[0]
    iota = jax.lax.iota(jnp.int32, l.shape[0])
    i0 = jnp.argmax(l).astype(jnp.int32)
    l2 = jnp.where(iota == i0, -jnp.inf, l)
    i1 = jnp.argmax(l2).astype(jnp.int32)
    idx_ref[0] = i0
    idx_ref[1] = i1


def _block_body(idx_ref, h_ref, g_ref, beta_ref, w1_ref, b1_ref, w2_ref,
                b2_ref, t_ref, o_ref, w1g_ref, w2s_ref, b1p_ref, c2_ref):
    t = pl.program_id(0)
    D = h_ref.shape[1]

    @pl.when(t == 0)
    def _():
        g_col = g_ref[0].reshape(D, 1)
        C = D // 4
        for c in range(4):
            lo, hi = c * C, (c + 1) * C
            w1g_ref[lo:hi, :] = (g_col[lo:hi, :]
                                 * w1_ref[0, lo:hi, :]).astype(jnp.bfloat16)
            w2s_ref[lo:hi, :] = (0.3 * w2_ref[0, lo:hi, :]).astype(jnp.bfloat16)
        b1p_ref[...] = (jnp.dot(beta_ref[0], w1_ref[0],
                                preferred_element_type=jnp.float32)
                        + b1_ref[0])
        c2_ref[...] = 0.3 * b2_ref[0]

    h = h_ref[...]
    mu = jnp.mean(h, axis=1, keepdims=True)
    var = jnp.mean((h - mu) ** 2, axis=1, keepdims=True)
    z = ((h - mu) * jax.lax.rsqrt(var + 1e-5)).astype(jnp.bfloat16)
    a = jnp.tanh(jnp.dot(z, w1g_ref[...],
                         preferred_element_type=jnp.float32) + b1p_ref[...])
    q = jnp.dot(a.astype(jnp.bfloat16), w2s_ref[...],
                preferred_element_type=jnp.float32) + c2_ref[...]
    o_ref[...] = h + q * (1.0 + 0.1 * t_ref[0])


def _block_pass(h, top_idx, step, ln_g, ln_beta, W1, b1, W2, b2, torsion,
                rows_per_tile):
    BS, D = h.shape
    S_per_batch = BS // torsion.shape[0]
    grid = BS // rows_per_tile

    def widx3(t, s):
        del t
        return (s[step], 0, 0)

    NB = ln_g.shape[0]
    B = torsion.shape[0]
    return pl.pallas_call(
        _block_body,
        grid_spec=pltpu.PrefetchScalarGridSpec(
            num_scalar_prefetch=1,
            grid=(grid,),
            in_specs=[
                pl.BlockSpec((rows_per_tile, D), lambda t, s: (t, 0)),
                pl.BlockSpec((1, 1, D), widx3),
                pl.BlockSpec((1, 1, D), widx3),
                pl.BlockSpec((1, D, D), widx3),
                pl.BlockSpec((1, 1, D), widx3),
                pl.BlockSpec((1, D, D), widx3),
                pl.BlockSpec((1, 1, D), widx3),
                pl.BlockSpec((1, 1, D),
                             lambda t, s: (t * rows_per_tile // S_per_batch, 0, 0)),
            ],
            out_specs=pl.BlockSpec((rows_per_tile, D), lambda t, s: (t, 0)),
            scratch_shapes=[
                pltpu.VMEM((D, D), jnp.bfloat16),
                pltpu.VMEM((D, D), jnp.bfloat16),
                pltpu.VMEM((1, D), jnp.float32),
                pltpu.VMEM((1, D), jnp.float32),
            ],
        ),
        out_shape=jax.ShapeDtypeStruct((BS, D), jnp.float32),
        compiler_params=pltpu.CompilerParams(
            dimension_semantics=("arbitrary",),
        ),
    )(top_idx, h, ln_g.reshape(NB, 1, D), ln_beta.reshape(NB, 1, D), W1,
      b1.reshape(NB, 1, D), W2, b2.reshape(NB, 1, D), torsion.reshape(B, 1, D))


def kernel(embodied_input, disembodied_input, torsion_field, sel_W, sel_b,
           ln_g, ln_beta, W1, b1, W2, b2, max_active_blocks):
    B, S, D = embodied_input.shape
    NB = sel_b.shape[0]

    x0 = embodied_input[0]
    top_idx = pl.pallas_call(
        _selector_body,
        in_specs=[
            pl.BlockSpec(memory_space=pltpu.VMEM),
            pl.BlockSpec(memory_space=pltpu.VMEM),
            pl.BlockSpec(memory_space=pltpu.VMEM),
        ],
        out_specs=pl.BlockSpec(memory_space=pltpu.SMEM),
        out_shape=jax.ShapeDtypeStruct((2,), jnp.int32),
    )(x0, sel_W, sel_b.reshape(1, NB))

    h = embodied_input.reshape(B * S, D)
    for i in range(2):
        h = _block_pass(h, top_idx, i, ln_g, ln_beta, W1, b1, W2, b2,
                        torsion_field, rows_per_tile=1024)
    return h.reshape(B, S, D)


# speedup vs baseline: 1.2745x; 1.0511x over previous
import jax
import jax.numpy as jnp
from jax.experimental import pallas as pl
from jax.experimental.pallas import tpu as pltpu


def _selector_body(x_ref, w_ref, b_ref, idx_ref):
    pooled = jnp.mean(x_ref[...], axis=0, keepdims=True)
    logits = jnp.dot(pooled, w_ref[...],
                     preferred_element_type=jnp.float32) + b_ref[...]
    l = logits[0]
    iota = jax.lax.iota(jnp.int32, l.shape[0])
    i0 = jnp.argmax(l).astype(jnp.int32)
    l2 = jnp.where(iota == i0, -jnp.inf, l)
    i1 = jnp.argmax(l2).astype(jnp.int32)
    idx_ref[0] = i0
    idx_ref[1] = i1


def _fold(g_ref, beta_ref, w1_ref, b1_ref, w2_ref, b2_ref,
          w1g_ref, w2s_ref, b1p_ref, c2_ref, D):
    g_col = g_ref[0].reshape(D, 1)
    C = D // 4
    for c in range(4):
        lo, hi = c * C, (c + 1) * C
        w1g_ref[lo:hi, :] = (g_col[lo:hi, :]
                             * w1_ref[0, lo:hi, :]).astype(jnp.bfloat16)
        w2s_ref[lo:hi, :] = (0.3 * w2_ref[0, lo:hi, :]).astype(jnp.bfloat16)
    b1p_ref[...] = (jnp.dot(beta_ref[0], w1_ref[0],
                            preferred_element_type=jnp.float32) + b1_ref[0])
    c2_ref[...] = 0.3 * b2_ref[0]


def _apply_block(h, w1g_ref, w2s_ref, b1p_ref, c2_ref, tt):
    mu = jnp.mean(h, axis=1, keepdims=True)
    var = jnp.mean((h - mu) ** 2, axis=1, keepdims=True)
    z = ((h - mu) * jax.lax.rsqrt(var + 1e-5)).astype(jnp.bfloat16)
    a = jnp.tanh(jnp.dot(z, w1g_ref[...],
                         preferred_element_type=jnp.float32) + b1p_ref[...])
    q = jnp.dot(a.astype(jnp.bfloat16), w2s_ref[...],
                preferred_element_type=jnp.float32) + c2_ref[...]
    return h + q * tt


def _chain_body(idx_ref, h_ref,
                ga_ref, beta_a_ref, w1a_ref, b1a_ref, w2a_ref, b2a_ref,
                gb_ref, beta_b_ref, w1b_ref, b1b_ref, w2b_ref, b2b_ref,
                t_ref, o_ref,
                w1g_a, w2s_a, b1p_a, c2_a, w1g_b, w2s_b, b1p_b, c2_b):
    t = pl.program_id(0)
    D = h_ref.shape[1]

    @pl.when(t == 0)
    def _():
        _fold(ga_ref, beta_a_ref, w1a_ref, b1a_ref, w2a_ref, b2a_ref,
              w1g_a, w2s_a, b1p_a, c2_a, D)
        _fold(gb_ref, beta_b_ref, w1b_ref, b1b_ref, w2b_ref, b2b_ref,
              w1g_b, w2s_b, b1p_b, c2_b, D)

    tt = 1.0 + 0.1 * t_ref[0]
    h = h_ref[...]
    h = _apply_block(h, w1g_a, w2s_a, b1p_a, c2_a, tt)
    h = _apply_block(h, w1g_b, w2s_b, b1p_b, c2_b, tt)
    o_ref[...] = h


def kernel(embodied_input, disembodied_input, torsion_field, sel_W, sel_b,
           ln_g, ln_beta, W1, b1, W2, b2, max_active_blocks):
    B, S, D = embodied_input.shape
    NB = sel_b.shape[0]
    BS = B * S

    x0 = embodied_input[0]
    top_idx = pl.pallas_call(
        _selector_body,
        in_specs=[
            pl.BlockSpec(memory_space=pltpu.VMEM),
            pl.BlockSpec(memory_space=pltpu.VMEM),
            pl.BlockSpec(memory_space=pltpu.VMEM),
        ],
        out_specs=pl.BlockSpec(memory_space=pltpu.SMEM),
        out_shape=jax.ShapeDtypeStruct((2,), jnp.int32),
    )(x0, sel_W, sel_b.reshape(1, NB))

    R = 1024
    T = BS // R
    S_per_batch = S

    def widx3(step):
        def f(t, s):
            del t
            return (s[step], 0, 0)
        return f

    def wspec(step, shape):
        return pl.BlockSpec(shape, widx3(step),
                            pipeline_mode=pl.Buffered(buffer_count=1))

    g3 = ln_g.reshape(NB, 1, D)
    be3 = ln_beta.reshape(NB, 1, D)
    b13 = b1.reshape(NB, 1, D)
    b23 = b2.reshape(NB, 1, D)

    h = pl.pallas_call(
        _chain_body,
        grid_spec=pltpu.PrefetchScalarGridSpec(
            num_scalar_prefetch=1,
            grid=(T,),
            in_specs=[
                pl.BlockSpec((R, D), lambda t, s: (t, 0)),
                wspec(0, (1, 1, D)), wspec(0, (1, 1, D)),
                wspec(0, (1, D, D)), wspec(0, (1, 1, D)),
                wspec(0, (1, D, D)), wspec(0, (1, 1, D)),
                wspec(1, (1, 1, D)), wspec(1, (1, 1, D)),
                wspec(1, (1, D, D)), wspec(1, (1, 1, D)),
                wspec(1, (1, D, D)), wspec(1, (1, 1, D)),
                pl.BlockSpec((1, 1, D),
                             lambda t, s: (t * R // S_per_batch, 0, 0)),
            ],
            out_specs=pl.BlockSpec((R, D), lambda t, s: (t, 0)),
            scratch_shapes=[
                pltpu.VMEM((D, D), jnp.bfloat16),
                pltpu.VMEM((D, D), jnp.bfloat16),
                pltpu.VMEM((1, D), jnp.float32),
                pltpu.VMEM((1, D), jnp.float32),
                pltpu.VMEM((D, D), jnp.bfloat16),
                pltpu.VMEM((D, D), jnp.bfloat16),
                pltpu.VMEM((1, D), jnp.float32),
                pltpu.VMEM((1, D), jnp.float32),
            ],
        ),
        out_shape=jax.ShapeDtypeStruct((BS, D), jnp.float32),
        compiler_params=pltpu.CompilerParams(
            dimension_semantics=("arbitrary",),
        ),
    )(top_idx, embodied_input.reshape(BS, D),
      g3, be3, W1, b13, W2, b23,
      g3, be3, W1, b13, W2, b23,
      torsion_field.reshape(B, 1, D))
    return h.reshape(B, S, D)
